# baseline (speedup 1.0000x reference)
import functools

import jax
import jax.numpy as jnp
import numpy as np
from jax import lax
from jax.experimental import pallas as pl
from jax.experimental.pallas import tpu as pltpu
from jax.experimental.pallas import tpu_sc as plsc

NC, NS, L = 2, 16, 16
C = 20000
U = 10
S = 16
V = 100000
NEG = np.float32(-3.4e38)
BIG = np.int32(1 << 20)


def _clamp01(x):
    return jnp.minimum(jnp.maximum(x, 0), 1)


def _sc_body(ids_hbm, logits_hbm, prev_hbm,
             draft_hbm, counts_hbm, last_hbm,
             b0, b1, ids_v, pv_v, g_v, out_v, s0, s1):
    nch = V // C
    tot = S * nch
    w = lax.axis_index("s") * NC + lax.axis_index("c")
    base = w * S * V
    li = lax.iota(jnp.int32, L)

    def src(t):
        r = t // nch
        c = t - r * nch
        return logits_hbm.at[pl.ds(base + r * V + c * C, C)], r, c

    pltpu.make_async_copy(src(0)[0], b0, s0).start()

    def rot(x, sh):
        dn = lax.GatherDimensionNumbers(
            offset_dims=(), collapsed_slice_dims=(0,), start_index_map=(0,))
        return lax.gather(x, ((li + sh) % L)[:, None], dn, slice_sizes=(1,),
                          mode=lax.GatherScatterMode.PROMISE_IN_BOUNDS)

    def step(t, bufA, semA, bufB, semB, carry):
        m, mi = carry

        @pl.when(t + 1 < tot)
        def _():
            pltpu.make_async_copy(src(t + 1)[0], bufB, semB).start()

        sT, r, c = src(t)
        pltpu.make_async_copy(sT, bufA, semA).wait()
        cb = c * C

        def inner(j, car):
            im, imi = car
            for u in range(U):
                off = j * (L * U) + u * L
                v = bufA[pl.ds(off, L)]
                pred = v > im
                idx = li + (cb + off)
                im = jnp.where(pred, v, im)
                imi = jnp.where(pred, idx, imi)
            return im, imi

        m, mi = lax.fori_loop(0, C // (L * U), inner, (m, mi))

        mv = m
        for sh in (1, 2, 4, 8):
            mv = jnp.maximum(mv, rot(mv, sh))
        neq01 = jnp.minimum((mv - m) * np.float32(1e38), np.float32(1.0))
        mi_f = mi.astype(jnp.float32)
        cand = mi_f + neq01 * (np.float32(BIG) - mi_f)
        for sh in (1, 2, 4, 8):
            cand = jnp.minimum(cand, rot(cand, sh))
        g_v[pl.ds(r, L)] = cand.astype(jnp.int32)

        ki = c // (nch - 1)
        kf = ki.astype(jnp.float32)
        m = m * (1.0 - kf) + NEG * kf
        mi = mi * (1 - ki)
        return m, mi

    def gbody(g, carry):
        carry = step(g * 2, b0, s0, b1, s1, carry)
        carry = step(g * 2 + 1, b1, s1, b0, s0, carry)
        return carry

    init = (jnp.full((L,), NEG), jnp.zeros((L,), jnp.int32))
    lax.fori_loop(0, tot // 2, gbody, init)

    pltpu.sync_copy(prev_hbm, pv_v)
    pa = pv_v[pl.ds(0, L)]
    pb = pv_v[pl.ds(L, L)]
    eqa = 1 - jnp.minimum(jnp.abs(li - w), 1)
    eqb = 1 - jnp.minimum(jnp.abs(li - (w - L)), 1)
    p = pa * eqa + pb * eqb
    for sh in (1, 2, 4, 8):
        p = p + rot(p, sh)

    pltpu.sync_copy(ids_hbm.at[pl.ds(w * S, S)], ids_v)
    greedy = g_v[pl.ds(0, L)]
    shifted = rot(ids_v[...], 1)
    neq = jnp.minimum(jnp.abs(greedy - shifted), 1)
    ge01 = _clamp01(li - p + 2)
    le01 = _clamp01((S - 2) - li + 1)
    act = ge01 * le01 * neq
    mism = (S - 1) + act * (li - (S - 1))
    first = mism
    for sh in (1, 2, 4, 8):
        first = jnp.minimum(first, rot(first, sh))
    nn = jnp.maximum(first - (p - 1), 0)
    na = p + nn

    c0 = 1 - jnp.minimum(li, 1)
    c1 = 1 - jnp.minimum(jnp.abs(li - 1), 1)
    out_v[...] = nn * c0 + na * c1
    pltpu.sync_copy(out_v, counts_hbm.at[pl.ds(w * L, L)])
    na_scalar = out_v[...][1]

    a01 = _clamp01(na - 1 - li)
    b01 = 1 - jnp.minimum(jnp.abs(li - (na - 1)), 1)
    draft = shifted * a01 + greedy * b01
    out_v[...] = draft
    pltpu.sync_copy(out_v, draft_hbm.at[pl.ds(w * L, L)])

    g_off = base + (na_scalar - 1) * V
    pltpu.make_async_copy(logits_hbm.at[pl.ds(g_off, C)], b0, s0).start()
    for c in range(nch):
        bufA, semA = (b0, s0) if c % 2 == 0 else (b1, s1)
        bufB, semB = (b1, s1) if c % 2 == 0 else (b0, s0)
        if c + 1 < nch:
            pltpu.make_async_copy(
                logits_hbm.at[pl.ds(g_off + (c + 1) * C, C)], bufB, semB).start()
        pltpu.make_async_copy(
            logits_hbm.at[pl.ds(g_off + c * C, C)], bufA, semA).wait()
        pltpu.sync_copy(bufA, last_hbm.at[pl.ds(w * V + c * C, C)])


def kernel(input_ids, target_logits, num_previously_accepted):
    B, S_ = input_ids.shape
    V_ = target_logits.shape[2]
    logits1d = target_logits.reshape(-1)
    ids1d = input_ids.reshape(-1)
    prev = num_previously_accepted.astype(jnp.int32)

    mesh = plsc.VectorSubcoreMesh(core_axis_name="c", subcore_axis_name="s",
                                  num_cores=NC, num_subcores=NS)
    f = pl.kernel(
        _sc_body,
        out_type=[
            jax.ShapeDtypeStruct((B * L,), jnp.int32),
            jax.ShapeDtypeStruct((B * L,), jnp.int32),
            jax.ShapeDtypeStruct((B * V_,), jnp.float32),
        ],
        mesh=mesh,
        scratch_types=[
            pltpu.VMEM((C,), jnp.float32),
            pltpu.VMEM((C,), jnp.float32),
            pltpu.VMEM((L,), jnp.int32),
            pltpu.VMEM((2 * L,), jnp.int32),
            pltpu.VMEM((2 * L,), jnp.int32),
            pltpu.VMEM((L,), jnp.int32),
            pltpu.SemaphoreType.DMA,
            pltpu.SemaphoreType.DMA,
        ],
    )
    draft, counts, last = f(ids1d, logits1d, prev)

    draft2 = draft.reshape(B, L)
    counts2 = counts.reshape(B, L)
    draft_input_ids = draft2[:, :S_].astype(input_ids.dtype)
    num_newly = counts2[:, 0].astype(num_previously_accepted.dtype)
    num_accepted = counts2[:, 1].astype(num_previously_accepted.dtype)
    return (draft_input_ids, num_newly, num_accepted,
            last.reshape(B, 1, V_))
